# baseline (speedup 1.0000x reference)
import functools

import jax
import jax.numpy as jnp
from jax import lax
from jax.experimental import pallas as pl
from jax.experimental.pallas import tpu as pltpu
from jax.experimental.pallas import tpu_sc as plsc

NC, NS, L = 2, 16, 16
NW = NC * NS

BATCH, FIELDS, D = 4096, 26, 32
R = BATCH * FIELDS
RPW = R // NW
CHUNK = 128
NCHUNK = RPW // CHUNK
TQ = 1000000 * D // 128
OROWS = R * D // 128
ORPW = OROWS // NW
OCH = ORPW // NCHUNK


def _rsqrt(v):
    i = lax.bitcast_convert_type(v, jnp.int32)
    y = lax.bitcast_convert_type(jnp.int32(0x5F3759DF) - (i >> 1), jnp.float32)
    for _ in range(3):
        y = y * (1.5 - 0.5 * v * y * y)
    return y


_mesh = plsc.VectorSubcoreMesh(core_axis_name="c", subcore_axis_name="s")


@functools.partial(
    pl.kernel,
    out_type=jax.ShapeDtypeStruct((OROWS, 128), jnp.float32),
    mesh=_mesh,
    scratch_types=[
        pltpu.VMEM((NCHUNK, CHUNK), jnp.int32),
        pltpu.VMEM((NCHUNK, CHUNK), jnp.int32),
        pltpu.VMEM((2, CHUNK, 128), jnp.float32),
        pltpu.VMEM((2, OCH, 128), jnp.float32),
        pltpu.VMEM((D,), jnp.float32),
        pltpu.VMEM((D,), jnp.float32),
        pltpu.SemaphoreType.DMA,
        pltpu.SemaphoreType.DMA,
    ],
)
def _embed_ln(x_hbm, table_hbm, gamma_hbm, beta_hbm, out_hbm,
              idx_v, off_v, big_v, outbuf, gamma_v, beta_v, gsem, osem):
    wid = lax.axis_index("s") * NC + lax.axis_index("c")

    pltpu.sync_copy(x_hbm.at[wid], idx_v)
    pltpu.sync_copy(gamma_hbm, gamma_v)
    pltpu.sync_copy(beta_hbm, beta_v)

    def prep_body(c, _):
        for k in range(CHUNK // L):
            v = idx_v[c, pl.ds(k * L, L)]
            off_v[c, pl.ds(k * L, L)] = (v & 3) << 5
            idx_v[c, pl.ds(k * L, L)] = v >> 2
        return 0

    lax.fori_loop(0, NCHUNK, prep_body, 0)

    g_lo = gamma_v[pl.ds(0, L)]
    g_hi = gamma_v[pl.ds(L, L)]
    b_lo = beta_v[pl.ds(0, L)]
    b_hi = beta_v[pl.ds(L, L)]

    lane = lax.iota(jnp.int32, L)
    perms = [lane ^ (1 << k) for k in range(4)]
    _dnums = lax.GatherDimensionNumbers(
        offset_dims=(), collapsed_slice_dims=(0,), start_index_map=(0,))

    def lane_perm(v, p):
        return lax.gather(v, p[:, None], _dnums, (1,),
                          mode=lax.GatherScatterMode.PROMISE_IN_BOUNDS)

    def allreduce_sum(v):
        for p in perms:
            v = v + lane_perm(v, p)
        return v

    def fire_gather(c, buf):
        pltpu.make_async_copy(
            table_hbm.at[idx_v.at[c]], big_v.at[buf], gsem).start()

    fire_gather(0, 0)
    fire_gather(1, 1)

    def chunk_body(c, _):
        buf = c & 1
        pltpu.make_async_copy(
            table_hbm.at[pl.ds(0, CHUNK)], big_v.at[buf], gsem).wait()

        @pl.when(c >= 2)
        def _():
            pltpu.make_async_copy(
                out_hbm.at[pl.ds(0, OCH)], outbuf.at[buf], osem).wait()

        def row_block(i, _):
            offs = off_v[c, pl.ds(i * L, L)]
            for j in range(L):
                r = i * L + j
                s = offs[j]
                a = big_v[buf, r, pl.ds(s, L)]
                bb = big_v[buf, r, pl.ds(s + L, L)]
                total = allreduce_sum(a + bb)
                total2 = allreduce_sum(a * a + bb * bb)
                mean = total * (1.0 / D)
                var = total2 * (1.0 / D) - mean * mean
                rstd = _rsqrt(var + 1e-5)
                orow = i * 4 + j // 4
                ocol = (j % 4) * D
                outbuf[buf, orow, pl.ds(ocol, L)] = (a - mean) * rstd * g_lo + b_lo
                outbuf[buf, orow, pl.ds(ocol + L, L)] = (bb - mean) * rstd * g_hi + b_hi
            return 0

        lax.fori_loop(0, CHUNK // L, row_block, 0)

        @pl.when(c + 2 < NCHUNK)
        def _():
            fire_gather(c + 2, buf)

        pltpu.make_async_copy(
            outbuf.at[buf],
            out_hbm.at[pl.ds(wid * ORPW + c * OCH, OCH)],
            osem).start()
        return 0

    lax.fori_loop(0, NCHUNK, chunk_body, 0)

    pltpu.make_async_copy(out_hbm.at[pl.ds(0, OCH)], outbuf.at[0], osem).wait()
    pltpu.make_async_copy(out_hbm.at[pl.ds(0, OCH)], outbuf.at[1], osem).wait()


def kernel(x, table, gamma, beta):
    x3d = x.reshape(NW, NCHUNK, CHUNK)
    t128 = table.reshape(TQ, 128)
    out = _embed_ln(x3d, t128, gamma, beta)
    return out.reshape(BATCH, FIELDS * D)
